# Initial kernel scaffold; baseline (speedup 1.0000x reference)
import jax
import jax.numpy as jnp
from jax.experimental import pallas as pl


def kernel(x, edge_index, batch, params):
    raise NotImplementedError("write your pallas kernel here")


# speedup vs baseline: 2.4852x; 2.4852x over previous
import functools

import jax
import jax.numpy as jnp
import numpy as np
from jax import lax
from jax.experimental import pallas as pl
from jax.experimental.pallas import tpu as pltpu
from jax.experimental.pallas import tpu_sc as plsc

_N = 10000
_NPAD = 10112
_E = 320000
_NC, _NS = 2, 16
_NW = _NC * _NS
_CH = 128
_K = 80
_EPAD = _NW * _K * _CH
_RPT = _NPAD // _NS
_DS = 64
_DR = 128
_NG = 64


@functools.cache
def _sc_mesh():
    return plsc.VectorSubcoreMesh(core_axis_name="c", subcore_axis_name="s",
                                  num_cores=_NC, num_subcores=_NS)


def _spmm_body(p_hbm, src_hbm, dst_hbm, zero_hbm, out_hbm,
               src_v, dst_v, rows_v, sem, acc_sh):
    cid = lax.axis_index("c")
    sid = lax.axis_index("s")
    wid = sid * _NC + cid
    r0 = sid * _RPT
    pltpu.sync_copy(zero_hbm.at[pl.ds(r0, _RPT)], acc_sh.at[pl.ds(r0, _RPT)])
    plsc.subcore_barrier()

    def chunk(k, carry):
        pltpu.sync_copy(src_hbm.at[wid, k], src_v)
        pltpu.sync_copy(dst_hbm.at[wid, k], dst_v)
        pltpu.async_copy(p_hbm.at[src_v], rows_v, sem).wait()
        pltpu.sync_copy(rows_v, acc_sh.at[dst_v], add=True)
        return carry

    lax.fori_loop(0, _K, chunk, 0)
    plsc.subcore_barrier()
    pltpu.sync_copy(acc_sh.at[pl.ds(r0, _RPT)],
                    out_hbm.at[cid, pl.ds(r0, _RPT)])


@functools.cache
def _spmm():
    return pl.kernel(
        _spmm_body,
        out_type=jax.ShapeDtypeStruct((_NC, _NPAD, _DR), jnp.float32),
        mesh=_sc_mesh(),
        scratch_types=[
            pltpu.VMEM((_CH,), jnp.int32),
            pltpu.VMEM((_CH,), jnp.int32),
            pltpu.VMEM((_CH, _DR), jnp.float32),
            pltpu.SemaphoreType.DMA,
            pltpu.VMEM_SHARED((_NPAD, _DR), jnp.float32),
        ],
    )


def _gelu(x):
    return x * 0.5 * (1.0 + lax.erf(x * np.float32(1.0 / np.sqrt(2.0))))


def _ln(h, g, b):
    mu = jnp.mean(h, axis=-1, keepdims=True)
    var = jnp.mean((h - mu) ** 2, axis=-1, keepdims=True)
    return (h - mu) / jnp.sqrt(var + 1e-5) * g + b


def _pre_body(x_ref, wlT_ref, wrT_ref, p_ref, r_ref):
    x = x_ref[...]
    col = lax.broadcasted_iota(jnp.int32, (_NPAD, _DR), 1)
    p_ref[...] = (jnp.dot(x, wlT_ref[...], preferred_element_type=jnp.float32)
                  + (col == _DS).astype(jnp.float32))
    r_ref[...] = jnp.dot(x, wrT_ref[...], preferred_element_type=jnp.float32)


def _post_first_body(agg_ref, r_ref, bl_ref, g_ref, b_ref,
                     wlT_ref, h_ref, p_ref, inv_ref):
    a2 = agg_ref[...]
    agg = a2[0, :, 0:_DS] + a2[1, :, 0:_DS]
    cnt = a2[0, :, _DS:_DS + 1] + a2[1, :, _DS:_DS + 1]
    inv = 1.0 / jnp.maximum(cnt, 1.0)
    inv_ref[...] = inv
    t = agg * inv + bl_ref[...] + r_ref[...]
    f = _ln(_gelu(t), g_ref[...], b_ref[...])
    h_ref[...] = f
    p_ref[...] = jnp.dot(f, wlT_ref[...], preferred_element_type=jnp.float32)


def _post_mid_body(agg_ref, inv_ref, h_ref, wrT_ref, bl_ref, g_ref, b_ref,
                   wlT_ref, hn_ref, p_ref):
    a2 = agg_ref[...]
    agg = a2[0, :, 0:_DS] + a2[1, :, 0:_DS]
    h = h_ref[...]
    t = (agg * inv_ref[...] + bl_ref[...]
         + jnp.dot(h, wrT_ref[...], preferred_element_type=jnp.float32))
    f = _ln(_gelu(t), g_ref[...], b_ref[...]) + h
    hn_ref[...] = f
    p_ref[...] = jnp.dot(f, wlT_ref[...], preferred_element_type=jnp.float32)


def _post_last_body(agg_ref, inv_ref, h_ref, wrT_ref, bl_ref, g_ref, b_ref,
                    hn_ref):
    a2 = agg_ref[...]
    agg = a2[0, :, 0:_DS] + a2[1, :, 0:_DS]
    h = h_ref[...]
    t = (agg * inv_ref[...] + bl_ref[...]
         + jnp.dot(h, wrT_ref[...], preferred_element_type=jnp.float32))
    hn_ref[...] = _ln(_gelu(t), g_ref[...], b_ref[...]) + h


def _head_body(h_ref, batch_ref,
               w0, b0, g0, be0, w1, b1, g1, be1, w2, b2, g2, be2,
               w3, b3, g3, be3, wo, bo, out_ref):
    h = h_ref[...]
    bvec = batch_ref[...]
    gid = lax.broadcasted_iota(jnp.int32, (_NG, _NPAD), 0)
    onehot = (gid == bvec).astype(jnp.float32)
    m = jnp.dot(onehot, h, preferred_element_type=jnp.float32)
    for i, (w, b, g, be) in enumerate(
            ((w0, b0, g0, be0), (w1, b1, g1, be1),
             (w2, b2, g2, be2), (w3, b3, g3, be3))):
        f = jnp.dot(m, w[...], preferred_element_type=jnp.float32) + b[...]
        f = _ln(_gelu(f), g[...], be[...])
        m = f + m if i > 0 else f
    out_ref[...] = jnp.dot(m, wo[...], preferred_element_type=jnp.float32) + bo[...]


def _tc_call(body, out_shapes):
    return pl.pallas_call(body, out_shape=out_shapes)


def kernel(x, edge_index, batch, params):
    f32 = jnp.float32
    src = edge_index[0].astype(jnp.int32)
    dst = edge_index[1].astype(jnp.int32)
    pad = _EPAD - _E
    src3 = jnp.concatenate([src, jnp.zeros((pad,), jnp.int32)]).reshape(_NW, _K, _CH)
    dst3 = jnp.concatenate([dst, jnp.full((pad,), _N, jnp.int32)]).reshape(_NW, _K, _CH)
    x_pad = jnp.pad(x.astype(f32), ((0, _NPAD - _N), (0, 0)))
    batch_pad = jnp.pad(batch.astype(jnp.int32), (0, _NPAD - _N),
                        constant_values=_NG).reshape(1, _NPAD)
    zeros128 = jnp.zeros((_NPAD, _DR), f32)

    def row(p, name):
        return p[name].astype(f32).reshape(1, -1)

    def wlT_pad(i):
        w = params[f"sage{i}_Wl"].astype(f32).T
        return jnp.pad(w, ((0, 0), (0, _DR - _DS)))

    p0, r0 = _tc_call(_pre_body, (
        jax.ShapeDtypeStruct((_NPAD, _DR), f32),
        jax.ShapeDtypeStruct((_NPAD, _DS), f32),
    ))(x_pad, wlT_pad(0), params["sage0_Wr"].astype(f32).T)

    h2 = jax.ShapeDtypeStruct((_NPAD, _DS), f32)
    p2 = jax.ShapeDtypeStruct((_NPAD, _DR), f32)
    i2 = jax.ShapeDtypeStruct((_NPAD, 1), f32)
    agg2 = _spmm()(p0, src3, dst3, zeros128)
    h, p, inv = _tc_call(_post_first_body, (h2, p2, i2))(
        agg2, r0, row(params, "sage0_bl"), row(params, "sage0_g"),
        row(params, "sage0_b"), wlT_pad(1))

    for i in range(1, 7):
        agg2 = _spmm()(p, src3, dst3, zeros128)
        common = (agg2, inv, h, params[f"sage{i}_Wr"].astype(f32).T,
                  row(params, f"sage{i}_bl"), row(params, f"sage{i}_g"),
                  row(params, f"sage{i}_b"))
        if i < 6:
            h, p = _tc_call(_post_mid_body, (h2, p2))(*common, wlT_pad(i + 1))
        else:
            h = _tc_call(_post_last_body, h2)(*common)

    head_args = [h, batch_pad]
    for i in range(4):
        head_args += [params[f"mlp{i}_W"].astype(f32).T,
                      row(params, f"mlp{i}_b"), row(params, f"mlp{i}_g"),
                      row(params, f"mlp{i}_be")]
    head_args += [params["out_W"].astype(f32).T,
                  params["out_b"].astype(f32).reshape(1, 1)]
    out = _tc_call(_head_body, jax.ShapeDtypeStruct((_NG, 1), f32))(*head_args)
    return out


# speedup vs baseline: 5.0807x; 2.0444x over previous
import functools

import jax
import jax.numpy as jnp
import numpy as np
from jax import lax
from jax.experimental import pallas as pl
from jax.experimental.pallas import tpu as pltpu
from jax.experimental.pallas import tpu_sc as plsc

_N = 10000
_NPAD = 10112
_E = 320000
_NC, _NS = 2, 16
_NW = _NC * _NS
_CH = 128
_K = 80
_EPAD = _NW * _K * _CH
_RPT = _NPAD // _NS
_DS = 64
_NG = 64

_SC_PARAMS = pltpu.CompilerParams(use_tc_tiling_on_sc=False)


@functools.cache
def _sc_mesh():
    return plsc.VectorSubcoreMesh(core_axis_name="c", subcore_axis_name="s",
                                  num_cores=_NC, num_subcores=_NS)


_NBUF = 4
_KB = _K // _NBUF


def _spmm_body(p_hbm, src_hbm, dst_hbm, zero_hbm, out_hbm,
               srci_v, dsti_v, rows0, rows1, rows2, rows3,
               g0, g1, g2, g3, s0, s1, s2, s3, acc_sh):
    rows = (rows0, rows1, rows2, rows3)
    gsem = (g0, g1, g2, g3)
    ssem = (s0, s1, s2, s3)
    cid = lax.axis_index("c")
    sid = lax.axis_index("s")
    wid = sid * _NC + cid
    r0 = sid * _RPT
    pltpu.sync_copy(zero_hbm.at[pl.ds(r0, _RPT)], acc_sh.at[pl.ds(r0, _RPT)])
    pltpu.sync_copy(src_hbm.at[wid], srci_v)
    pltpu.sync_copy(dst_hbm.at[wid], dsti_v)
    plsc.subcore_barrier()

    for b in range(_NBUF):
        pltpu.async_copy(p_hbm.at[srci_v.at[b]], rows[b], gsem[b])

    def outer(g, carry):
        for b in range(_NBUF):
            k = g * _NBUF + b
            pltpu.make_async_copy(p_hbm.at[srci_v.at[k]], rows[b],
                                  gsem[b]).wait()
            pltpu.async_copy(rows[b], acc_sh.at[dsti_v.at[k]], ssem[b],
                             add=True)
        for b in range(_NBUF):
            k = g * _NBUF + b
            pltpu.make_async_copy(rows[b], acc_sh.at[dsti_v.at[k]],
                                  ssem[b]).wait()

            @pl.when(g < _KB - 1)
            def _():
                pltpu.async_copy(p_hbm.at[srci_v.at[k + _NBUF]], rows[b],
                                 gsem[b])
        return carry

    lax.fori_loop(0, _KB, outer, 0)
    plsc.subcore_barrier()
    pltpu.sync_copy(acc_sh.at[pl.ds(r0, _RPT)],
                    out_hbm.at[cid, pl.ds(r0, _RPT)])


@functools.cache
def _spmm():
    return pl.kernel(
        _spmm_body,
        out_type=jax.ShapeDtypeStruct((_NC, _NPAD, _DS), jnp.float32),
        mesh=_sc_mesh(),
        compiler_params=_SC_PARAMS,
        scratch_types=[
            pltpu.VMEM((_K, _CH), jnp.int32),
            pltpu.VMEM((_K, _CH), jnp.int32),
        ] + [pltpu.VMEM((_CH, _DS), jnp.float32)] * _NBUF
          + [pltpu.SemaphoreType.DMA] * (2 * _NBUF)
          + [pltpu.VMEM_SHARED((_NPAD, _DS), jnp.float32)],
    )


def _cnt_body(dst_hbm, ones_hbm, zero_hbm, out_hbm, dsti_v, ones_v, sem,
              acc_sh):
    cid = lax.axis_index("c")
    sid = lax.axis_index("s")
    wid = sid * _NC + cid
    r0 = sid * _RPT
    pltpu.sync_copy(zero_hbm.at[pl.ds(r0, _RPT)], acc_sh.at[pl.ds(r0, _RPT)])
    pltpu.sync_copy(dst_hbm.at[wid], dsti_v)
    pltpu.sync_copy(ones_hbm, ones_v)
    plsc.subcore_barrier()

    def fire(k, carry):
        pltpu.async_copy(ones_v, acc_sh.at[dsti_v.at[k]], sem, add=True)
        return carry

    lax.fori_loop(0, _K, fire, 0)

    def drain(k, carry):
        pltpu.make_async_copy(ones_v, acc_sh.at[dsti_v.at[0]], sem).wait()
        return carry

    lax.fori_loop(0, _K, drain, 0)
    plsc.subcore_barrier()
    pltpu.sync_copy(acc_sh.at[pl.ds(r0, _RPT)],
                    out_hbm.at[cid, pl.ds(r0, _RPT)])


@functools.cache
def _cnt():
    return pl.kernel(
        _cnt_body,
        out_type=jax.ShapeDtypeStruct((_NC, _NPAD, _DS), jnp.float32),
        mesh=_sc_mesh(),
        compiler_params=_SC_PARAMS,
        scratch_types=[
            pltpu.VMEM((_K, _CH), jnp.int32),
            pltpu.VMEM((_CH, _DS), jnp.float32),
            pltpu.SemaphoreType.DMA,
            pltpu.VMEM_SHARED((_NPAD, _DS), jnp.float32),
        ],
    )


def _gelu(x):
    return x * 0.5 * (1.0 + lax.erf(x * np.float32(1.0 / np.sqrt(2.0))))


def _ln(h, g, b):
    mu = jnp.mean(h, axis=-1, keepdims=True)
    var = jnp.mean((h - mu) ** 2, axis=-1, keepdims=True)
    return (h - mu) / jnp.sqrt(var + 1e-5) * g + b


def _pre_body(x_ref, wlT_ref, wrT_ref, cnt_ref, p_ref, r_ref, inv_ref):
    x = x_ref[...]
    p_ref[...] = jnp.dot(x, wlT_ref[...], preferred_element_type=jnp.float32)
    r_ref[...] = jnp.dot(x, wrT_ref[...], preferred_element_type=jnp.float32)
    cnt = cnt_ref[...]
    c = cnt[0, :, 0:1] + cnt[1, :, 0:1]
    inv_ref[...] = 1.0 / jnp.maximum(c, 1.0)


def _post_first_body(agg_ref, inv_ref, r_ref, bl_ref, g_ref, b_ref,
                     wlT_ref, h_ref, p_ref):
    a2 = agg_ref[...]
    agg = a2[0] + a2[1]
    t = agg * inv_ref[...] + bl_ref[...] + r_ref[...]
    f = _ln(_gelu(t), g_ref[...], b_ref[...])
    h_ref[...] = f
    p_ref[...] = jnp.dot(f, wlT_ref[...], preferred_element_type=jnp.float32)


def _post_mid_body(agg_ref, inv_ref, h_ref, wrT_ref, bl_ref, g_ref, b_ref,
                   wlT_ref, hn_ref, p_ref):
    a2 = agg_ref[...]
    agg = a2[0] + a2[1]
    h = h_ref[...]
    t = (agg * inv_ref[...] + bl_ref[...]
         + jnp.dot(h, wrT_ref[...], preferred_element_type=jnp.float32))
    f = _ln(_gelu(t), g_ref[...], b_ref[...]) + h
    hn_ref[...] = f
    p_ref[...] = jnp.dot(f, wlT_ref[...], preferred_element_type=jnp.float32)


def _post_last_body(agg_ref, inv_ref, h_ref, wrT_ref, bl_ref, g_ref, b_ref,
                    hn_ref):
    a2 = agg_ref[...]
    agg = a2[0] + a2[1]
    h = h_ref[...]
    t = (agg * inv_ref[...] + bl_ref[...]
         + jnp.dot(h, wrT_ref[...], preferred_element_type=jnp.float32))
    hn_ref[...] = _ln(_gelu(t), g_ref[...], b_ref[...]) + h


def _head_body(h_ref, batch_ref,
               w0, b0, g0, be0, w1, b1, g1, be1, w2, b2, g2, be2,
               w3, b3, g3, be3, wo, bo, out_ref):
    h = h_ref[...]
    bvec = batch_ref[...]
    gid = lax.broadcasted_iota(jnp.int32, (_NG, _NPAD), 0)
    onehot = (gid == bvec).astype(jnp.float32)
    m = jnp.dot(onehot, h, preferred_element_type=jnp.float32)
    for i, (w, b, g, be) in enumerate(
            ((w0, b0, g0, be0), (w1, b1, g1, be1),
             (w2, b2, g2, be2), (w3, b3, g3, be3))):
        f = jnp.dot(m, w[...], preferred_element_type=jnp.float32) + b[...]
        f = _ln(_gelu(f), g[...], be[...])
        m = f + m if i > 0 else f
    out_ref[...] = jnp.dot(m, wo[...], preferred_element_type=jnp.float32) + bo[...]


def _tc_call(body, out_shapes):
    return pl.pallas_call(body, out_shape=out_shapes)


def kernel(x, edge_index, batch, params):
    f32 = jnp.float32
    src = edge_index[0].astype(jnp.int32)
    dst = edge_index[1].astype(jnp.int32)
    pad = _EPAD - _E
    src3 = jnp.concatenate([src, jnp.zeros((pad,), jnp.int32)]).reshape(_NW, _K, _CH)
    dst3 = jnp.concatenate([dst, jnp.full((pad,), _N, jnp.int32)]).reshape(_NW, _K, _CH)
    x_pad = jnp.pad(x.astype(f32), ((0, _NPAD - _N), (0, 0)))
    batch_pad = jnp.pad(batch.astype(jnp.int32), (0, _NPAD - _N),
                        constant_values=_NG).reshape(1, _NPAD)
    zeros64 = jnp.zeros((_NPAD, _DS), f32)
    ones64 = jnp.ones((_CH, _DS), f32)

    def row(p, name):
        return p[name].astype(f32).reshape(1, -1)

    cnt2 = _cnt()(dst3, ones64, zeros64)

    p0, r0, inv = _tc_call(_pre_body, (
        jax.ShapeDtypeStruct((_NPAD, _DS), f32),
        jax.ShapeDtypeStruct((_NPAD, _DS), f32),
        jax.ShapeDtypeStruct((_NPAD, 1), f32),
    ))(x_pad, params["sage0_Wl"].astype(f32).T, params["sage0_Wr"].astype(f32).T,
       cnt2)

    h2 = jax.ShapeDtypeStruct((_NPAD, _DS), f32)
    agg2 = _spmm()(p0, src3, dst3, zeros64)
    h, p = _tc_call(_post_first_body, (h2, h2))(
        agg2, inv, r0, row(params, "sage0_bl"), row(params, "sage0_g"),
        row(params, "sage0_b"), params["sage1_Wl"].astype(f32).T)

    for i in range(1, 7):
        agg2 = _spmm()(p, src3, dst3, zeros64)
        common = (agg2, inv, h, params[f"sage{i}_Wr"].astype(f32).T,
                  row(params, f"sage{i}_bl"), row(params, f"sage{i}_g"),
                  row(params, f"sage{i}_b"))
        if i < 6:
            h, p = _tc_call(_post_mid_body, (h2, h2))(
                *common, params[f"sage{i + 1}_Wl"].astype(f32).T)
        else:
            h = _tc_call(_post_last_body, h2)(*common)

    head_args = [h, batch_pad]
    for i in range(4):
        head_args += [params[f"mlp{i}_W"].astype(f32).T,
                      row(params, f"mlp{i}_b"), row(params, f"mlp{i}_g"),
                      row(params, f"mlp{i}_be")]
    head_args += [params["out_W"].astype(f32).T,
                  params["out_b"].astype(f32).reshape(1, 1)]
    out = _tc_call(_head_body, jax.ShapeDtypeStruct((_NG, 1), f32))(*head_args)
    return out


# speedup vs baseline: 5.1708x; 1.0177x over previous
import functools

import jax
import jax.numpy as jnp
import numpy as np
from jax import lax
from jax.experimental import pallas as pl
from jax.experimental.pallas import tpu as pltpu
from jax.experimental.pallas import tpu_sc as plsc

_N = 10000
_NPAD = 10112
_E = 320000
_NC, _NS = 2, 16
_NW = _NC * _NS
_CH = 128
_K = 80
_EPAD = _NW * _K * _CH
_RPT = _NPAD // _NS
_DS = 64
_NG = 64

_SC_PARAMS = pltpu.CompilerParams(use_tc_tiling_on_sc=False)


@functools.cache
def _sc_mesh():
    return plsc.VectorSubcoreMesh(core_axis_name="c", subcore_axis_name="s",
                                  num_cores=_NC, num_subcores=_NS)


_NBUF = 8
_KB = _K // _NBUF


def _spmm_body(p_hbm, src_hbm, dst_hbm, zero_hbm, out_hbm,
               srci_v, dsti_v, rows0, rows1, rows2, rows3, rows4, rows5,
               rows6, rows7, g0, g1, g2, g3, g4, g5, g6, g7,
               s0, s1, s2, s3, s4, s5, s6, s7, acc_sh):
    rows = (rows0, rows1, rows2, rows3, rows4, rows5, rows6, rows7)
    gsem = (g0, g1, g2, g3, g4, g5, g6, g7)
    ssem = (s0, s1, s2, s3, s4, s5, s6, s7)
    cid = lax.axis_index("c")
    sid = lax.axis_index("s")
    wid = sid * _NC + cid
    r0 = sid * _RPT
    pltpu.sync_copy(zero_hbm.at[pl.ds(r0, _RPT)], acc_sh.at[pl.ds(r0, _RPT)])
    pltpu.sync_copy(src_hbm.at[wid], srci_v)
    pltpu.sync_copy(dst_hbm.at[wid], dsti_v)
    plsc.subcore_barrier()

    for b in range(_NBUF):
        pltpu.async_copy(p_hbm.at[srci_v.at[b]], rows[b], gsem[b])

    def outer(g, carry):
        for b in range(_NBUF):
            k = g * _NBUF + b
            pltpu.make_async_copy(p_hbm.at[srci_v.at[k]], rows[b],
                                  gsem[b]).wait()
            pltpu.async_copy(rows[b], acc_sh.at[dsti_v.at[k]], ssem[b],
                             add=True)
        for b in range(_NBUF):
            k = g * _NBUF + b
            pltpu.make_async_copy(rows[b], acc_sh.at[dsti_v.at[k]],
                                  ssem[b]).wait()

            @pl.when(g < _KB - 1)
            def _():
                pltpu.async_copy(p_hbm.at[srci_v.at[k + _NBUF]], rows[b],
                                 gsem[b])
        return carry

    lax.fori_loop(0, _KB, outer, 0)
    plsc.subcore_barrier()
    pltpu.sync_copy(acc_sh.at[pl.ds(r0, _RPT)],
                    out_hbm.at[cid, pl.ds(r0, _RPT)])


@functools.cache
def _spmm():
    return pl.kernel(
        _spmm_body,
        out_type=jax.ShapeDtypeStruct((_NC, _NPAD, _DS), jnp.float32),
        mesh=_sc_mesh(),
        compiler_params=_SC_PARAMS,
        scratch_types=[
            pltpu.VMEM((_K, _CH), jnp.int32),
            pltpu.VMEM((_K, _CH), jnp.int32),
        ] + [pltpu.VMEM((_CH, _DS), jnp.float32)] * _NBUF
          + [pltpu.SemaphoreType.DMA] * (2 * _NBUF)
          + [pltpu.VMEM_SHARED((_NPAD, _DS), jnp.float32)],
    )


def _cnt_body(dst_hbm, ones_hbm, zero_hbm, out_hbm, dsti_v, ones_v, sem,
              acc_sh):
    cid = lax.axis_index("c")
    sid = lax.axis_index("s")
    wid = sid * _NC + cid
    r0 = sid * _RPT
    pltpu.sync_copy(zero_hbm.at[pl.ds(r0, _RPT)], acc_sh.at[pl.ds(r0, _RPT)])
    pltpu.sync_copy(dst_hbm.at[wid], dsti_v)
    pltpu.sync_copy(ones_hbm, ones_v)
    plsc.subcore_barrier()

    def fire(k, carry):
        pltpu.async_copy(ones_v, acc_sh.at[dsti_v.at[k]], sem, add=True)
        return carry

    lax.fori_loop(0, _K, fire, 0)

    def drain(k, carry):
        pltpu.make_async_copy(ones_v, acc_sh.at[dsti_v.at[0]], sem).wait()
        return carry

    lax.fori_loop(0, _K, drain, 0)
    plsc.subcore_barrier()
    pltpu.sync_copy(acc_sh.at[pl.ds(r0, _RPT)],
                    out_hbm.at[cid, pl.ds(r0, _RPT)])


@functools.cache
def _cnt():
    return pl.kernel(
        _cnt_body,
        out_type=jax.ShapeDtypeStruct((_NC, _NPAD, _DS), jnp.float32),
        mesh=_sc_mesh(),
        compiler_params=_SC_PARAMS,
        scratch_types=[
            pltpu.VMEM((_K, _CH), jnp.int32),
            pltpu.VMEM((_CH, _DS), jnp.float32),
            pltpu.SemaphoreType.DMA,
            pltpu.VMEM_SHARED((_NPAD, _DS), jnp.float32),
        ],
    )


def _gelu(x):
    return x * 0.5 * (1.0 + lax.erf(x * np.float32(1.0 / np.sqrt(2.0))))


def _ln(h, g, b):
    mu = jnp.mean(h, axis=-1, keepdims=True)
    var = jnp.mean((h - mu) ** 2, axis=-1, keepdims=True)
    return (h - mu) / jnp.sqrt(var + 1e-5) * g + b


def _pre_body(x_ref, wlT_ref, wrT_ref, cnt_ref, p_ref, r_ref, inv_ref):
    x = x_ref[...]
    p_ref[...] = jnp.dot(x, wlT_ref[...], preferred_element_type=jnp.float32)
    r_ref[...] = jnp.dot(x, wrT_ref[...], preferred_element_type=jnp.float32)
    cnt = cnt_ref[...]
    c = cnt[0, :, 0:1] + cnt[1, :, 0:1]
    inv_ref[...] = 1.0 / jnp.maximum(c, 1.0)


def _post_first_body(agg_ref, inv_ref, r_ref, bl_ref, g_ref, b_ref,
                     wlT_ref, h_ref, p_ref):
    a2 = agg_ref[...]
    agg = a2[0] + a2[1]
    t = agg * inv_ref[...] + bl_ref[...] + r_ref[...]
    f = _ln(_gelu(t), g_ref[...], b_ref[...])
    h_ref[...] = f
    p_ref[...] = jnp.dot(f, wlT_ref[...], preferred_element_type=jnp.float32)


def _post_mid_body(agg_ref, inv_ref, h_ref, wrT_ref, bl_ref, g_ref, b_ref,
                   wlT_ref, hn_ref, p_ref):
    a2 = agg_ref[...]
    agg = a2[0] + a2[1]
    h = h_ref[...]
    t = (agg * inv_ref[...] + bl_ref[...]
         + jnp.dot(h, wrT_ref[...], preferred_element_type=jnp.float32))
    f = _ln(_gelu(t), g_ref[...], b_ref[...]) + h
    hn_ref[...] = f
    p_ref[...] = jnp.dot(f, wlT_ref[...], preferred_element_type=jnp.float32)


def _post_last_body(agg_ref, inv_ref, h_ref, wrT_ref, bl_ref, g_ref, b_ref,
                    hn_ref):
    a2 = agg_ref[...]
    agg = a2[0] + a2[1]
    h = h_ref[...]
    t = (agg * inv_ref[...] + bl_ref[...]
         + jnp.dot(h, wrT_ref[...], preferred_element_type=jnp.float32))
    hn_ref[...] = _ln(_gelu(t), g_ref[...], b_ref[...]) + h


def _head_body(h_ref, batch_ref,
               w0, b0, g0, be0, w1, b1, g1, be1, w2, b2, g2, be2,
               w3, b3, g3, be3, wo, bo, out_ref):
    h = h_ref[...]
    bvec = batch_ref[...]
    gid = lax.broadcasted_iota(jnp.int32, (_NG, _NPAD), 0)
    onehot = (gid == bvec).astype(jnp.float32)
    m = jnp.dot(onehot, h, preferred_element_type=jnp.float32)
    for i, (w, b, g, be) in enumerate(
            ((w0, b0, g0, be0), (w1, b1, g1, be1),
             (w2, b2, g2, be2), (w3, b3, g3, be3))):
        f = jnp.dot(m, w[...], preferred_element_type=jnp.float32) + b[...]
        f = _ln(_gelu(f), g[...], be[...])
        m = f + m if i > 0 else f
    out_ref[...] = jnp.dot(m, wo[...], preferred_element_type=jnp.float32) + bo[...]


def _tc_call(body, out_shapes):
    return pl.pallas_call(body, out_shape=out_shapes)


def kernel(x, edge_index, batch, params):
    f32 = jnp.float32
    src = edge_index[0].astype(jnp.int32)
    dst = edge_index[1].astype(jnp.int32)
    pad = _EPAD - _E
    src3 = jnp.concatenate([src, jnp.zeros((pad,), jnp.int32)]).reshape(_NW, _K, _CH)
    dst3 = jnp.concatenate([dst, jnp.full((pad,), _N, jnp.int32)]).reshape(_NW, _K, _CH)
    x_pad = jnp.pad(x.astype(f32), ((0, _NPAD - _N), (0, 0)))
    batch_pad = jnp.pad(batch.astype(jnp.int32), (0, _NPAD - _N),
                        constant_values=_NG).reshape(1, _NPAD)
    zeros64 = jnp.zeros((_NPAD, _DS), f32)
    ones64 = jnp.ones((_CH, _DS), f32)

    def row(p, name):
        return p[name].astype(f32).reshape(1, -1)

    cnt2 = _cnt()(dst3, ones64, zeros64)

    p0, r0, inv = _tc_call(_pre_body, (
        jax.ShapeDtypeStruct((_NPAD, _DS), f32),
        jax.ShapeDtypeStruct((_NPAD, _DS), f32),
        jax.ShapeDtypeStruct((_NPAD, 1), f32),
    ))(x_pad, params["sage0_Wl"].astype(f32).T, params["sage0_Wr"].astype(f32).T,
       cnt2)

    h2 = jax.ShapeDtypeStruct((_NPAD, _DS), f32)
    agg2 = _spmm()(p0, src3, dst3, zeros64)
    h, p = _tc_call(_post_first_body, (h2, h2))(
        agg2, inv, r0, row(params, "sage0_bl"), row(params, "sage0_g"),
        row(params, "sage0_b"), params["sage1_Wl"].astype(f32).T)

    for i in range(1, 7):
        agg2 = _spmm()(p, src3, dst3, zeros64)
        common = (agg2, inv, h, params[f"sage{i}_Wr"].astype(f32).T,
                  row(params, f"sage{i}_bl"), row(params, f"sage{i}_g"),
                  row(params, f"sage{i}_b"))
        if i < 6:
            h, p = _tc_call(_post_mid_body, (h2, h2))(
                *common, params[f"sage{i + 1}_Wl"].astype(f32).T)
        else:
            h = _tc_call(_post_last_body, h2)(*common)

    head_args = [h, batch_pad]
    for i in range(4):
        head_args += [params[f"mlp{i}_W"].astype(f32).T,
                      row(params, f"mlp{i}_b"), row(params, f"mlp{i}_g"),
                      row(params, f"mlp{i}_be")]
    head_args += [params["out_W"].astype(f32).T,
                  params["out_b"].astype(f32).reshape(1, 1)]
    out = _tc_call(_head_body, jax.ShapeDtypeStruct((_NG, 1), f32))(*head_args)
    return out


# speedup vs baseline: 9.2892x; 1.7965x over previous
import functools

import jax
import jax.numpy as jnp
import numpy as np
from jax import lax
from jax.experimental import pallas as pl
from jax.experimental.pallas import tpu as pltpu
from jax.experimental.pallas import tpu_sc as plsc

_N = 10000
_NPAD = 10112
_E = 320000
_NC, _NS = 2, 16
_NW = _NC * _NS
_CH = 128
_K = 81
_EPAD = _NW * _K * _CH
_RPT = _NPAD // _NS
_DS = 64
_NG = 64

_SC_PARAMS = pltpu.CompilerParams(use_tc_tiling_on_sc=False)


@functools.cache
def _sc_mesh():
    return plsc.VectorSubcoreMesh(core_axis_name="c", subcore_axis_name="s",
                                  num_cores=_NC, num_subcores=_NS)


_NBUF = 3
_KB = _K // _NBUF


def _spmm_body(p_hbm, src_hbm, dst_hbm, zero_hbm, out_hbm,
               srci_v, dsti_v, rows0, rows1, rows2,
               g0, g1, g2, s0, s1, s2, acc_sh, p_sh):
    rows = (rows0, rows1, rows2)
    gsem = (g0, g1, g2)
    ssem = (s0, s1, s2)
    cid = lax.axis_index("c")
    sid = lax.axis_index("s")
    wid = sid * _NC + cid
    r0 = sid * _RPT
    pltpu.sync_copy(zero_hbm.at[pl.ds(r0, _RPT)], acc_sh.at[pl.ds(r0, _RPT)])
    pltpu.sync_copy(p_hbm.at[pl.ds(r0, _RPT)], p_sh.at[pl.ds(r0, _RPT)])
    pltpu.sync_copy(src_hbm.at[wid], srci_v)
    pltpu.sync_copy(dst_hbm.at[wid], dsti_v)
    plsc.subcore_barrier()

    for b in range(_NBUF):
        pltpu.async_copy(p_sh.at[srci_v.at[b]], rows[b], gsem[b])

    def outer(g, carry):
        for b in range(_NBUF):
            k = g * _NBUF + b
            pltpu.make_async_copy(p_sh.at[srci_v.at[k]], rows[b],
                                  gsem[b]).wait()
            pltpu.async_copy(rows[b], acc_sh.at[dsti_v.at[k]], ssem[b],
                             add=True)
        for b in range(_NBUF):
            k = g * _NBUF + b
            pltpu.make_async_copy(rows[b], acc_sh.at[dsti_v.at[k]],
                                  ssem[b]).wait()

            @pl.when(g < _KB - 1)
            def _():
                pltpu.async_copy(p_sh.at[srci_v.at[k + _NBUF]], rows[b],
                                 gsem[b])
        return carry

    lax.fori_loop(0, _KB, outer, 0)
    plsc.subcore_barrier()
    pltpu.sync_copy(acc_sh.at[pl.ds(r0, _RPT)],
                    out_hbm.at[cid, pl.ds(r0, _RPT)])


@functools.cache
def _spmm():
    return pl.kernel(
        _spmm_body,
        out_type=jax.ShapeDtypeStruct((_NC, _NPAD, _DS), jnp.float32),
        mesh=_sc_mesh(),
        compiler_params=_SC_PARAMS,
        scratch_types=[
            pltpu.VMEM((_K, _CH), jnp.int32),
            pltpu.VMEM((_K, _CH), jnp.int32),
        ] + [pltpu.VMEM((_CH, _DS), jnp.float32)] * _NBUF
          + [pltpu.SemaphoreType.DMA] * (2 * _NBUF)
          + [pltpu.VMEM_SHARED((_NPAD, _DS), jnp.float32)] * 2,
    )


def _cnt_body(dst_hbm, ones_hbm, zero_hbm, out_hbm, dsti_v, ones_v, sem,
              acc_sh):
    cid = lax.axis_index("c")
    sid = lax.axis_index("s")
    wid = sid * _NC + cid
    r0 = sid * _RPT
    pltpu.sync_copy(zero_hbm.at[pl.ds(r0, _RPT)], acc_sh.at[pl.ds(r0, _RPT)])
    pltpu.sync_copy(dst_hbm.at[wid], dsti_v)
    pltpu.sync_copy(ones_hbm, ones_v)
    plsc.subcore_barrier()

    def fire(k, carry):
        pltpu.async_copy(ones_v, acc_sh.at[dsti_v.at[k]], sem, add=True)
        return carry

    lax.fori_loop(0, _K, fire, 0)

    def drain(k, carry):
        pltpu.make_async_copy(ones_v, acc_sh.at[dsti_v.at[0]], sem).wait()
        return carry

    lax.fori_loop(0, _K, drain, 0)
    plsc.subcore_barrier()
    pltpu.sync_copy(acc_sh.at[pl.ds(r0, _RPT)],
                    out_hbm.at[cid, pl.ds(r0, _RPT)])


@functools.cache
def _cnt():
    return pl.kernel(
        _cnt_body,
        out_type=jax.ShapeDtypeStruct((_NC, _NPAD, _DS), jnp.float32),
        mesh=_sc_mesh(),
        compiler_params=_SC_PARAMS,
        scratch_types=[
            pltpu.VMEM((_K, _CH), jnp.int32),
            pltpu.VMEM((_CH, _DS), jnp.float32),
            pltpu.SemaphoreType.DMA,
            pltpu.VMEM_SHARED((_NPAD, _DS), jnp.float32),
        ],
    )


def _gelu(x):
    return x * 0.5 * (1.0 + lax.erf(x * np.float32(1.0 / np.sqrt(2.0))))


def _ln(h, g, b):
    mu = jnp.mean(h, axis=-1, keepdims=True)
    var = jnp.mean((h - mu) ** 2, axis=-1, keepdims=True)
    return (h - mu) / jnp.sqrt(var + 1e-5) * g + b


def _pre_body(x_ref, wlT_ref, wrT_ref, cnt_ref, p_ref, r_ref, inv_ref):
    x = x_ref[...]
    p_ref[...] = jnp.dot(x, wlT_ref[...], preferred_element_type=jnp.float32)
    r_ref[...] = jnp.dot(x, wrT_ref[...], preferred_element_type=jnp.float32)
    cnt = cnt_ref[...]
    c = cnt[0, :, 0:1] + cnt[1, :, 0:1]
    inv_ref[...] = 1.0 / jnp.maximum(c, 1.0)


def _post_first_body(agg_ref, inv_ref, r_ref, bl_ref, g_ref, b_ref,
                     wlT_ref, h_ref, p_ref):
    a2 = agg_ref[...]
    agg = a2[0] + a2[1]
    t = agg * inv_ref[...] + bl_ref[...] + r_ref[...]
    f = _ln(_gelu(t), g_ref[...], b_ref[...])
    h_ref[...] = f
    p_ref[...] = jnp.dot(f, wlT_ref[...], preferred_element_type=jnp.float32)


def _post_mid_body(agg_ref, inv_ref, h_ref, wrT_ref, bl_ref, g_ref, b_ref,
                   wlT_ref, hn_ref, p_ref):
    a2 = agg_ref[...]
    agg = a2[0] + a2[1]
    h = h_ref[...]
    t = (agg * inv_ref[...] + bl_ref[...]
         + jnp.dot(h, wrT_ref[...], preferred_element_type=jnp.float32))
    f = _ln(_gelu(t), g_ref[...], b_ref[...]) + h
    hn_ref[...] = f
    p_ref[...] = jnp.dot(f, wlT_ref[...], preferred_element_type=jnp.float32)


def _post_last_body(agg_ref, inv_ref, h_ref, wrT_ref, bl_ref, g_ref, b_ref,
                    hn_ref):
    a2 = agg_ref[...]
    agg = a2[0] + a2[1]
    h = h_ref[...]
    t = (agg * inv_ref[...] + bl_ref[...]
         + jnp.dot(h, wrT_ref[...], preferred_element_type=jnp.float32))
    hn_ref[...] = _ln(_gelu(t), g_ref[...], b_ref[...]) + h


def _head_body(h_ref, batch_ref,
               w0, b0, g0, be0, w1, b1, g1, be1, w2, b2, g2, be2,
               w3, b3, g3, be3, wo, bo, out_ref):
    h = h_ref[...]
    bvec = batch_ref[...]
    gid = lax.broadcasted_iota(jnp.int32, (_NG, _NPAD), 0)
    onehot = (gid == bvec).astype(jnp.float32)
    m = jnp.dot(onehot, h, preferred_element_type=jnp.float32)
    for i, (w, b, g, be) in enumerate(
            ((w0, b0, g0, be0), (w1, b1, g1, be1),
             (w2, b2, g2, be2), (w3, b3, g3, be3))):
        f = jnp.dot(m, w[...], preferred_element_type=jnp.float32) + b[...]
        f = _ln(_gelu(f), g[...], be[...])
        m = f + m if i > 0 else f
    out_ref[...] = jnp.dot(m, wo[...], preferred_element_type=jnp.float32) + bo[...]


def _tc_call(body, out_shapes):
    return pl.pallas_call(body, out_shape=out_shapes)


def kernel(x, edge_index, batch, params):
    f32 = jnp.float32
    src = edge_index[0].astype(jnp.int32)
    dst = edge_index[1].astype(jnp.int32)
    pad = _EPAD - _E
    src3 = jnp.concatenate([src, jnp.zeros((pad,), jnp.int32)]).reshape(_NW, _K, _CH)
    dst3 = jnp.concatenate([dst, jnp.full((pad,), _N, jnp.int32)]).reshape(_NW, _K, _CH)
    x_pad = jnp.pad(x.astype(f32), ((0, _NPAD - _N), (0, 0)))
    batch_pad = jnp.pad(batch.astype(jnp.int32), (0, _NPAD - _N),
                        constant_values=_NG).reshape(1, _NPAD)
    zeros64 = jnp.zeros((_NPAD, _DS), f32)
    ones64 = jnp.ones((_CH, _DS), f32)

    def row(p, name):
        return p[name].astype(f32).reshape(1, -1)

    cnt2 = _cnt()(dst3, ones64, zeros64)

    p0, r0, inv = _tc_call(_pre_body, (
        jax.ShapeDtypeStruct((_NPAD, _DS), f32),
        jax.ShapeDtypeStruct((_NPAD, _DS), f32),
        jax.ShapeDtypeStruct((_NPAD, 1), f32),
    ))(x_pad, params["sage0_Wl"].astype(f32).T, params["sage0_Wr"].astype(f32).T,
       cnt2)

    h2 = jax.ShapeDtypeStruct((_NPAD, _DS), f32)
    agg2 = _spmm()(p0, src3, dst3, zeros64)
    h, p = _tc_call(_post_first_body, (h2, h2))(
        agg2, inv, r0, row(params, "sage0_bl"), row(params, "sage0_g"),
        row(params, "sage0_b"), params["sage1_Wl"].astype(f32).T)

    for i in range(1, 7):
        agg2 = _spmm()(p, src3, dst3, zeros64)
        common = (agg2, inv, h, params[f"sage{i}_Wr"].astype(f32).T,
                  row(params, f"sage{i}_bl"), row(params, f"sage{i}_g"),
                  row(params, f"sage{i}_b"))
        if i < 6:
            h, p = _tc_call(_post_mid_body, (h2, h2))(
                *common, params[f"sage{i + 1}_Wl"].astype(f32).T)
        else:
            h = _tc_call(_post_last_body, h2)(*common)

    head_args = [h, batch_pad]
    for i in range(4):
        head_args += [params[f"mlp{i}_W"].astype(f32).T,
                      row(params, f"mlp{i}_b"), row(params, f"mlp{i}_g"),
                      row(params, f"mlp{i}_be")]
    head_args += [params["out_W"].astype(f32).T,
                  params["out_b"].astype(f32).reshape(1, 1)]
    out = _tc_call(_head_body, jax.ShapeDtypeStruct((_NG, 1), f32))(*head_args)
    return out


# speedup vs baseline: 10.0088x; 1.0775x over previous
import functools

import jax
import jax.numpy as jnp
import numpy as np
from jax import lax
from jax.experimental import pallas as pl
from jax.experimental.pallas import tpu as pltpu
from jax.experimental.pallas import tpu_sc as plsc

_N = 10000
_NPAD = 10112
_E = 320000
_NC, _NS = 2, 16
_NW = _NC * _NS
_CH = 128
_K = 81
_EPAD = _NW * _K * _CH
_RPT = _NPAD // _NS
_DS = 64
_NG = 64

_SC_PARAMS = pltpu.CompilerParams(use_tc_tiling_on_sc=False)


@functools.cache
def _sc_mesh():
    return plsc.VectorSubcoreMesh(core_axis_name="c", subcore_axis_name="s",
                                  num_cores=_NC, num_subcores=_NS)


_NBUF = 3
_KB = _K // _NBUF


def _spmm_body(p_hbm, src_hbm, dst_hbm, zero_hbm, out_hbm,
               srci_v, dsti_v, rows0, rows1, rows2,
               g0, g1, g2, s0, s1, s2, acc_sh, p_sh):
    rows = (rows0, rows1, rows2)
    gsem = (g0, g1, g2)
    ssem = (s0, s1, s2)
    cid = lax.axis_index("c")
    sid = lax.axis_index("s")
    wid = sid * _NC + cid
    r0 = sid * _RPT
    pltpu.sync_copy(zero_hbm.at[pl.ds(r0, _RPT)], acc_sh.at[pl.ds(r0, _RPT)])
    pltpu.sync_copy(p_hbm.at[pl.ds(r0, _RPT)], p_sh.at[pl.ds(r0, _RPT)])
    pltpu.sync_copy(src_hbm.at[wid], srci_v)
    pltpu.sync_copy(dst_hbm.at[wid], dsti_v)
    plsc.subcore_barrier()

    for b in range(_NBUF):
        pltpu.async_copy(p_sh.at[srci_v.at[b]], rows[b], gsem[b])

    def outer(g, carry):
        for b in range(_NBUF):
            k = g * _NBUF + b
            pltpu.make_async_copy(p_sh.at[srci_v.at[k]], rows[b],
                                  gsem[b]).wait()
            pltpu.async_copy(rows[b], acc_sh.at[dsti_v.at[k]], ssem[b],
                             add=True)
        for b in range(_NBUF):
            k = g * _NBUF + b
            pltpu.make_async_copy(rows[b], acc_sh.at[dsti_v.at[k]],
                                  ssem[b]).wait()

            @pl.when(g < _KB - 1)
            def _():
                pltpu.async_copy(p_sh.at[srci_v.at[k + _NBUF]], rows[b],
                                 gsem[b])
        return carry

    lax.fori_loop(0, _KB, outer, 0)
    plsc.subcore_barrier()
    pltpu.sync_copy(acc_sh.at[pl.ds(r0, _RPT)],
                    out_hbm.at[cid, pl.ds(r0, _RPT)])


@functools.cache
def _spmm():
    return pl.kernel(
        _spmm_body,
        out_type=jax.ShapeDtypeStruct((_NC, _NPAD, _DS), jnp.float32),
        mesh=_sc_mesh(),
        compiler_params=_SC_PARAMS,
        scratch_types=[
            pltpu.VMEM((_K, _CH), jnp.int32),
            pltpu.VMEM((_K, _CH), jnp.int32),
        ] + [pltpu.VMEM((_CH, _DS), jnp.float32)] * _NBUF
          + [pltpu.SemaphoreType.DMA] * (2 * _NBUF)
          + [pltpu.VMEM_SHARED((_NPAD, _DS), jnp.float32)] * 2,
    )


def _cnt_body(dst_hbm, ones_hbm, zero_hbm, out_hbm, dsti_v, ones_v, sem,
              acc_sh):
    cid = lax.axis_index("c")
    sid = lax.axis_index("s")
    wid = sid * _NC + cid
    r0 = sid * _RPT
    pltpu.sync_copy(zero_hbm.at[pl.ds(r0, _RPT)], acc_sh.at[pl.ds(r0, _RPT)])
    pltpu.sync_copy(dst_hbm.at[wid], dsti_v)
    pltpu.sync_copy(ones_hbm, ones_v)
    plsc.subcore_barrier()

    def fire(k, carry):
        pltpu.async_copy(ones_v, acc_sh.at[dsti_v.at[k]], sem, add=True)
        return carry

    lax.fori_loop(0, _K, fire, 0)

    def drain(k, carry):
        pltpu.make_async_copy(ones_v, acc_sh.at[dsti_v.at[0]], sem).wait()
        return carry

    lax.fori_loop(0, _K, drain, 0)
    plsc.subcore_barrier()
    pltpu.sync_copy(acc_sh.at[pl.ds(r0, _RPT)],
                    out_hbm.at[cid, pl.ds(r0, _RPT)])


@functools.cache
def _cnt():
    return pl.kernel(
        _cnt_body,
        out_type=jax.ShapeDtypeStruct((_NC, _NPAD, _DS), jnp.float32),
        mesh=_sc_mesh(),
        compiler_params=_SC_PARAMS,
        scratch_types=[
            pltpu.VMEM((_K, _CH), jnp.int32),
            pltpu.VMEM((_CH, _DS), jnp.float32),
            pltpu.SemaphoreType.DMA,
            pltpu.VMEM_SHARED((_NPAD, _DS), jnp.float32),
        ],
    )


def _gelu(x):
    return x * 0.5 * (1.0 + lax.erf(x * np.float32(1.0 / np.sqrt(2.0))))


def _ln(h, g, b):
    mu = jnp.mean(h, axis=-1, keepdims=True)
    var = jnp.mean((h - mu) ** 2, axis=-1, keepdims=True)
    return (h - mu) / jnp.sqrt(var + 1e-5) * g + b


def _pre_body(x_ref, wlT_ref, wrT_ref, cnt_ref, p_ref, r_ref, inv_ref):
    x = x_ref[...]
    p_ref[...] = jnp.dot(x, wlT_ref[...], preferred_element_type=jnp.float32)
    r_ref[...] = jnp.dot(x, wrT_ref[...], preferred_element_type=jnp.float32)
    cnt = cnt_ref[...]
    c = cnt[0, :, 0:1] + cnt[1, :, 0:1]
    inv_ref[...] = 1.0 / jnp.maximum(c, 1.0)


def _post_first_body(agg_ref, inv_ref, r_ref, bl_ref, g_ref, b_ref, h_ref):
    a2 = agg_ref[...]
    agg = a2[0] + a2[1]
    t = agg * inv_ref[...] + bl_ref[...] + r_ref[...]
    h_ref[...] = _ln(_gelu(t), g_ref[...], b_ref[...])


def _post_mid_body(agg_ref, inv_ref, h_ref, wlT_ref, wrT_ref, bl_ref, g_ref,
                   b_ref, hn_ref):
    a2 = agg_ref[...]
    mean = (a2[0] + a2[1]) * inv_ref[...]
    h = h_ref[...]
    t = (jnp.dot(mean, wlT_ref[...], preferred_element_type=jnp.float32)
         + bl_ref[...]
         + jnp.dot(h, wrT_ref[...], preferred_element_type=jnp.float32))
    hn_ref[...] = _ln(_gelu(t), g_ref[...], b_ref[...]) + h


def _head_body(h_ref, batch_ref,
               w0, b0, g0, be0, w1, b1, g1, be1, w2, b2, g2, be2,
               w3, b3, g3, be3, wo, bo, out_ref):
    h = h_ref[...]
    bvec = batch_ref[...]
    gid = lax.broadcasted_iota(jnp.int32, (_NG, _NPAD), 0)
    onehot = (gid == bvec).astype(jnp.float32)
    m = jnp.dot(onehot, h, preferred_element_type=jnp.float32)
    for i, (w, b, g, be) in enumerate(
            ((w0, b0, g0, be0), (w1, b1, g1, be1),
             (w2, b2, g2, be2), (w3, b3, g3, be3))):
        f = jnp.dot(m, w[...], preferred_element_type=jnp.float32) + b[...]
        f = _ln(_gelu(f), g[...], be[...])
        m = f + m if i > 0 else f
    out_ref[...] = jnp.dot(m, wo[...], preferred_element_type=jnp.float32) + bo[...]


def _tc_call(body, out_shapes):
    return pl.pallas_call(body, out_shape=out_shapes)


def kernel(x, edge_index, batch, params):
    f32 = jnp.float32
    src = edge_index[0].astype(jnp.int32)
    dst = edge_index[1].astype(jnp.int32)
    pad = _EPAD - _E
    src3 = jnp.concatenate([src, jnp.zeros((pad,), jnp.int32)]).reshape(_NW, _K, _CH)
    dst3 = jnp.concatenate([dst, jnp.full((pad,), _N, jnp.int32)]).reshape(_NW, _K, _CH)
    x_pad = jnp.pad(x.astype(f32), ((0, _NPAD - _N), (0, 0)))
    batch_pad = jnp.pad(batch.astype(jnp.int32), (0, _NPAD - _N),
                        constant_values=_NG).reshape(1, _NPAD)
    zeros64 = jnp.zeros((_NPAD, _DS), f32)
    ones64 = jnp.ones((_CH, _DS), f32)

    def row(p, name):
        return p[name].astype(f32).reshape(1, -1)

    cnt2 = _cnt()(dst3, ones64, zeros64)

    p0, r0, inv = _tc_call(_pre_body, (
        jax.ShapeDtypeStruct((_NPAD, _DS), f32),
        jax.ShapeDtypeStruct((_NPAD, _DS), f32),
        jax.ShapeDtypeStruct((_NPAD, 1), f32),
    ))(x_pad, params["sage0_Wl"].astype(f32).T, params["sage0_Wr"].astype(f32).T,
       cnt2)

    h2 = jax.ShapeDtypeStruct((_NPAD, _DS), f32)
    agg2 = _spmm()(p0, src3, dst3, zeros64)
    h = _tc_call(_post_first_body, h2)(
        agg2, inv, r0, row(params, "sage0_bl"), row(params, "sage0_g"),
        row(params, "sage0_b"))

    for i in range(1, 7):
        agg2 = _spmm()(h, src3, dst3, zeros64)
        h = _tc_call(_post_mid_body, h2)(
            agg2, inv, h, params[f"sage{i}_Wl"].astype(f32).T,
            params[f"sage{i}_Wr"].astype(f32).T,
            row(params, f"sage{i}_bl"), row(params, f"sage{i}_g"),
            row(params, f"sage{i}_b"))

    head_args = [h, batch_pad]
    for i in range(4):
        head_args += [params[f"mlp{i}_W"].astype(f32).T,
                      row(params, f"mlp{i}_b"), row(params, f"mlp{i}_g"),
                      row(params, f"mlp{i}_be")]
    head_args += [params["out_W"].astype(f32).T,
                  params["out_b"].astype(f32).reshape(1, 1)]
    out = _tc_call(_head_body, jax.ShapeDtypeStruct((_NG, 1), f32))(*head_args)
    return out


# speedup vs baseline: 10.4273x; 1.0418x over previous
import functools

import jax
import jax.numpy as jnp
import numpy as np
from jax import lax
from jax.experimental import pallas as pl
from jax.experimental.pallas import tpu as pltpu
from jax.experimental.pallas import tpu_sc as plsc

_N = 10000
_NPAD = 10112
_E = 320000
_NC, _NS = 2, 16
_NW = _NC * _NS
_CH = 64
_K = 162
_EPAD = _NW * _K * _CH
_RPT = _NPAD // _NS
_DS = 64
_NG = 64

_SC_PARAMS = pltpu.CompilerParams(use_tc_tiling_on_sc=False)


@functools.cache
def _sc_mesh():
    return plsc.VectorSubcoreMesh(core_axis_name="c", subcore_axis_name="s",
                                  num_cores=_NC, num_subcores=_NS)


_NBUF = 6
_KB = _K // _NBUF


def _spmm_body(p_hbm, src_hbm, dst_hbm, zero_hbm, out_hbm,
               srci_v, dsti_v, rows0, rows1, rows2, rows3, rows4, rows5,
               g0, g1, g2, g3, g4, g5, s0, s1, s2, s3, s4, s5, acc_sh, p_sh):
    rows = (rows0, rows1, rows2, rows3, rows4, rows5)
    gsem = (g0, g1, g2, g3, g4, g5)
    ssem = (s0, s1, s2, s3, s4, s5)
    cid = lax.axis_index("c")
    sid = lax.axis_index("s")
    wid = sid * _NC + cid
    r0 = sid * _RPT
    pltpu.sync_copy(zero_hbm.at[pl.ds(r0, _RPT)], acc_sh.at[pl.ds(r0, _RPT)])
    pltpu.sync_copy(p_hbm.at[pl.ds(r0, _RPT)], p_sh.at[pl.ds(r0, _RPT)])
    pltpu.sync_copy(src_hbm.at[wid], srci_v)
    pltpu.sync_copy(dst_hbm.at[wid], dsti_v)
    plsc.subcore_barrier()

    for b in range(_NBUF):
        pltpu.async_copy(p_sh.at[srci_v.at[b]], rows[b], gsem[b])

    def outer(g, carry):
        for b in range(_NBUF):
            k = g * _NBUF + b
            pltpu.make_async_copy(p_sh.at[srci_v.at[k]], rows[b],
                                  gsem[b]).wait()
            pltpu.async_copy(rows[b], acc_sh.at[dsti_v.at[k]], ssem[b],
                             add=True)
        for b in range(_NBUF):
            k = g * _NBUF + b
            pltpu.make_async_copy(rows[b], acc_sh.at[dsti_v.at[k]],
                                  ssem[b]).wait()

            @pl.when(g < _KB - 1)
            def _():
                pltpu.async_copy(p_sh.at[srci_v.at[k + _NBUF]], rows[b],
                                 gsem[b])
        return carry

    lax.fori_loop(0, _KB, outer, 0)
    plsc.subcore_barrier()
    pltpu.sync_copy(acc_sh.at[pl.ds(r0, _RPT)],
                    out_hbm.at[cid, pl.ds(r0, _RPT)])


@functools.cache
def _spmm():
    return pl.kernel(
        _spmm_body,
        out_type=jax.ShapeDtypeStruct((_NC, _NPAD, _DS), jnp.float32),
        mesh=_sc_mesh(),
        compiler_params=_SC_PARAMS,
        scratch_types=[
            pltpu.VMEM((_K, _CH), jnp.int32),
            pltpu.VMEM((_K, _CH), jnp.int32),
        ] + [pltpu.VMEM((_CH, _DS), jnp.float32)] * _NBUF
          + [pltpu.SemaphoreType.DMA] * (2 * _NBUF)
          + [pltpu.VMEM_SHARED((_NPAD, _DS), jnp.float32)] * 2,
    )


def _cnt_body(dst_hbm, ones_hbm, zero_hbm, out_hbm, dsti_v, ones_v, sem,
              acc_sh):
    cid = lax.axis_index("c")
    sid = lax.axis_index("s")
    wid = sid * _NC + cid
    r0 = sid * _RPT
    pltpu.sync_copy(zero_hbm.at[pl.ds(r0, _RPT)], acc_sh.at[pl.ds(r0, _RPT)])
    pltpu.sync_copy(dst_hbm.at[wid], dsti_v)
    pltpu.sync_copy(ones_hbm, ones_v)
    plsc.subcore_barrier()

    def fire(k, carry):
        pltpu.async_copy(ones_v, acc_sh.at[dsti_v.at[k]], sem, add=True)
        return carry

    lax.fori_loop(0, _K, fire, 0)

    def drain(k, carry):
        pltpu.make_async_copy(ones_v, acc_sh.at[dsti_v.at[0]], sem).wait()
        return carry

    lax.fori_loop(0, _K, drain, 0)
    plsc.subcore_barrier()
    pltpu.sync_copy(acc_sh.at[pl.ds(r0, _RPT)],
                    out_hbm.at[cid, pl.ds(r0, _RPT)])


@functools.cache
def _cnt():
    return pl.kernel(
        _cnt_body,
        out_type=jax.ShapeDtypeStruct((_NC, _NPAD, 8), jnp.float32),
        mesh=_sc_mesh(),
        compiler_params=_SC_PARAMS,
        scratch_types=[
            pltpu.VMEM((_K, _CH), jnp.int32),
            pltpu.VMEM((_CH, 8), jnp.float32),
            pltpu.SemaphoreType.DMA,
            pltpu.VMEM_SHARED((_NPAD, 8), jnp.float32),
        ],
    )


def _gelu(x):
    return x * 0.5 * (1.0 + lax.erf(x * np.float32(1.0 / np.sqrt(2.0))))


def _ln(h, g, b):
    mu = jnp.mean(h, axis=-1, keepdims=True)
    var = jnp.mean((h - mu) ** 2, axis=-1, keepdims=True)
    return (h - mu) / jnp.sqrt(var + 1e-5) * g + b


def _pre_body(x_ref, wlT_ref, wrT_ref, cnt_ref, p_ref, r_ref, inv_ref):
    x = x_ref[...]
    p_ref[...] = jnp.dot(x, wlT_ref[...], preferred_element_type=jnp.float32)
    r_ref[...] = jnp.dot(x, wrT_ref[...], preferred_element_type=jnp.float32)
    cnt = cnt_ref[...]
    c = cnt[0, :, 0:1] + cnt[1, :, 0:1]
    inv_ref[...] = 1.0 / jnp.maximum(c, 1.0)


def _post_first_body(agg_ref, inv_ref, r_ref, bl_ref, g_ref, b_ref, h_ref):
    a2 = agg_ref[...]
    agg = a2[0] + a2[1]
    t = agg * inv_ref[...] + bl_ref[...] + r_ref[...]
    h_ref[...] = _ln(_gelu(t), g_ref[...], b_ref[...])


def _post_mid_body(agg_ref, inv_ref, h_ref, wlT_ref, wrT_ref, bl_ref, g_ref,
                   b_ref, hn_ref):
    a2 = agg_ref[...]
    mean = (a2[0] + a2[1]) * inv_ref[...]
    h = h_ref[...]
    t = (jnp.dot(mean, wlT_ref[...], preferred_element_type=jnp.float32)
         + bl_ref[...]
         + jnp.dot(h, wrT_ref[...], preferred_element_type=jnp.float32))
    hn_ref[...] = _ln(_gelu(t), g_ref[...], b_ref[...]) + h


def _head_body(h_ref, batch_ref,
               w0, b0, g0, be0, w1, b1, g1, be1, w2, b2, g2, be2,
               w3, b3, g3, be3, wo, bo, out_ref):
    h = h_ref[...]
    bvec = batch_ref[...]
    gid = lax.broadcasted_iota(jnp.int32, (_NG, _NPAD), 0)
    onehot = (gid == bvec).astype(jnp.float32)
    m = jnp.dot(onehot, h, preferred_element_type=jnp.float32,
                precision=lax.Precision.HIGHEST)
    for i, (w, b, g, be) in enumerate(
            ((w0, b0, g0, be0), (w1, b1, g1, be1),
             (w2, b2, g2, be2), (w3, b3, g3, be3))):
        f = jnp.dot(m, w[...], preferred_element_type=jnp.float32) + b[...]
        f = _ln(_gelu(f), g[...], be[...])
        m = f + m if i > 0 else f
    out_ref[...] = jnp.dot(m, wo[...], preferred_element_type=jnp.float32) + bo[...]


def _tc_call(body, out_shapes):
    return pl.pallas_call(body, out_shape=out_shapes)


_NBLK = 8
_BR = _NPAD // _NBLK


def _tc_grid_call(body, in_specs, out_specs, out_shapes):
    return pl.pallas_call(
        body, grid=(_NBLK,),
        in_specs=in_specs, out_specs=out_specs, out_shape=out_shapes)


def _bs(shape, blocked_dim0=True):
    if blocked_dim0:
        if len(shape) == 2:
            return pl.BlockSpec((_BR, shape[1]), lambda i: (i, 0))
        return pl.BlockSpec((shape[0], _BR, shape[2]), lambda i: (0, i, 0))
    return pl.BlockSpec(shape, lambda i: tuple(0 for _ in shape))


def kernel(x, edge_index, batch, params):
    f32 = jnp.float32
    src = edge_index[0].astype(jnp.int32)
    dst = edge_index[1].astype(jnp.int32)
    pad = _EPAD - _E
    src3 = jnp.concatenate([src, jnp.zeros((pad,), jnp.int32)]).reshape(_NW, _K, _CH)
    dst3 = jnp.concatenate([dst, jnp.full((pad,), _N, jnp.int32)]).reshape(_NW, _K, _CH)
    x_pad = jnp.pad(x.astype(f32), ((0, _NPAD - _N), (0, 0)))
    batch_pad = jnp.pad(batch.astype(jnp.int32), (0, _NPAD - _N),
                        constant_values=_NG).reshape(1, _NPAD)
    zeros64 = jnp.zeros((_NPAD, _DS), f32)
    zeros8 = jnp.zeros((_NPAD, 8), f32)
    ones8 = jnp.ones((_CH, 8), f32)

    def row(p, name):
        return p[name].astype(f32).reshape(1, -1)

    cnt2 = _cnt()(dst3, ones8, zeros8)

    p0, r0, inv = _tc_grid_call(
        _pre_body,
        [_bs((_NPAD, 128)), _bs((128, _DS), False), _bs((128, _DS), False),
         _bs((2, _NPAD, 8))],
        (_bs((_NPAD, _DS)), _bs((_NPAD, _DS)), _bs((_NPAD, 1))),
        (jax.ShapeDtypeStruct((_NPAD, _DS), f32),
         jax.ShapeDtypeStruct((_NPAD, _DS), f32),
         jax.ShapeDtypeStruct((_NPAD, 1), f32)),
    )(x_pad, params["sage0_Wl"].astype(f32).T, params["sage0_Wr"].astype(f32).T,
      cnt2)

    h2 = jax.ShapeDtypeStruct((_NPAD, _DS), f32)
    agg2 = _spmm()(p0, src3, dst3, zeros64)
    h = _tc_grid_call(
        _post_first_body,
        [_bs((2, _NPAD, _DS)), _bs((_NPAD, 1)), _bs((_NPAD, _DS)),
         _bs((1, _DS), False), _bs((1, _DS), False), _bs((1, _DS), False)],
        _bs((_NPAD, _DS)), h2,
    )(agg2, inv, r0, row(params, "sage0_bl"), row(params, "sage0_g"),
      row(params, "sage0_b"))

    for i in range(1, 7):
        agg2 = _spmm()(h, src3, dst3, zeros64)
        h = _tc_grid_call(
            _post_mid_body,
            [_bs((2, _NPAD, _DS)), _bs((_NPAD, 1)), _bs((_NPAD, _DS)),
             _bs((_DS, _DS), False), _bs((_DS, _DS), False),
             _bs((1, _DS), False), _bs((1, _DS), False), _bs((1, _DS), False)],
            _bs((_NPAD, _DS)), h2,
        )(agg2, inv, h, params[f"sage{i}_Wl"].astype(f32).T,
          params[f"sage{i}_Wr"].astype(f32).T,
          row(params, f"sage{i}_bl"), row(params, f"sage{i}_g"),
          row(params, f"sage{i}_b"))

    head_args = [h, batch_pad]
    for i in range(4):
        head_args += [params[f"mlp{i}_W"].astype(f32).T,
                      row(params, f"mlp{i}_b"), row(params, f"mlp{i}_g"),
                      row(params, f"mlp{i}_be")]
    head_args += [params["out_W"].astype(f32).T,
                  params["out_b"].astype(f32).reshape(1, 1)]
    out = _tc_call(_head_body, jax.ShapeDtypeStruct((_NG, 1), f32))(*head_args)
    return out
